# baseline (speedup 1.0000x reference)
import functools

import jax
import jax.numpy as jnp
from jax import lax
from jax.experimental import pallas as pl
from jax.experimental.pallas import tpu as pltpu
from jax.experimental.pallas import tpu_sc as plsc

VOCAB = 100000
DIM = 128
BATCH = 16384
NEG = 5

_info = plsc.get_sparse_core_info()
_NC, _NS, _L = _info.num_cores, _info.num_subcores, _info.num_lanes
_NW = _NC * _NS
_BPW = BATCH // _NW
_CHUNK = 32
_NBUF = 4
_NCHUNKS = _BPW // _CHUNK
_NGROUPS = _CHUNK // _L

_mesh = plsc.VectorSubcoreMesh(core_axis_name="c", subcore_axis_name="s")


@functools.partial(
    pl.kernel,
    mesh=_mesh,
    compiler_params=pltpu.CompilerParams(needs_layout_passes=False),
    out_type=(
        jax.ShapeDtypeStruct((1, BATCH), jnp.float32),
        jax.ShapeDtypeStruct((NEG, BATCH), jnp.float32),
    ),
    scratch_types=[
        pltpu.VMEM((_BPW,), jnp.int32),
        pltpu.VMEM((_BPW,), jnp.int32),
        pltpu.VMEM((NEG, _BPW), jnp.int32),
        pltpu.VMEM((_NBUF * _CHUNK, DIM), jnp.float32),
        pltpu.VMEM((_NBUF * _CHUNK, DIM), jnp.float32),
        pltpu.VMEM((_NBUF * _CHUNK * NEG, DIM), jnp.float32),
        pltpu.VMEM((1, _BPW), jnp.float32),
        pltpu.VMEM((NEG, _BPW), jnp.float32),
        pltpu.SemaphoreType.DMA((_NBUF,)),
    ],
)
def _sc_scores(tgt_idx_hbm, ctx_idx_hbm, neg_idx_hbm, in_emb_hbm, out_emb_hbm,
               pos_out_hbm, neg_out_hbm,
               idx_t_v, idx_c_v, idx_n_v,
               tgt_v, ctx_v, neg_v,
               pos_s_v, neg_s_v, sems):
    wid = lax.axis_index("s") * _NC + lax.axis_index("c")
    base = wid * _BPW

    pltpu.sync_copy(tgt_idx_hbm.at[pl.ds(base, _BPW)], idx_t_v)
    pltpu.sync_copy(ctx_idx_hbm.at[pl.ds(base, _BPW)], idx_c_v)
    pltpu.sync_copy(neg_idx_hbm.at[:, pl.ds(base, _BPW)], idx_n_v)

    def fire(c):
        par = lax.rem(c, _NBUF)
        po = par * _CHUNK
        o = c * _CHUNK
        s_b = sems.at[par]
        pltpu.async_copy(
            in_emb_hbm.at[idx_t_v.at[pl.ds(o, _CHUNK)]],
            tgt_v.at[pl.ds(po, _CHUNK), :], s_b)
        pltpu.async_copy(
            out_emb_hbm.at[idx_c_v.at[pl.ds(o, _CHUNK)]],
            ctx_v.at[pl.ds(po, _CHUNK), :], s_b)
        for k in range(NEG):
            pltpu.async_copy(
                out_emb_hbm.at[idx_n_v.at[k, pl.ds(o, _CHUNK)]],
                neg_v.at[pl.ds(po * NEG + k * _CHUNK, _CHUNK), :], s_b)

    def wait(c):
        par = lax.rem(c, _NBUF)
        po = par * _CHUNK
        s_b = sems.at[par]
        pltpu.make_async_copy(
            in_emb_hbm.at[idx_t_v.at[pl.ds(0, _CHUNK)]],
            tgt_v.at[pl.ds(po, _CHUNK), :], s_b).wait()
        pltpu.make_async_copy(
            out_emb_hbm.at[idx_c_v.at[pl.ds(0, _CHUNK)]],
            ctx_v.at[pl.ds(po, _CHUNK), :], s_b).wait()
        for k in range(NEG):
            pltpu.make_async_copy(
                out_emb_hbm.at[idx_n_v.at[k, pl.ds(0, _CHUNK)]],
                neg_v.at[pl.ds(po * NEG + k * _CHUNK, _CHUNK), :], s_b).wait()

    lanes = lax.iota(jnp.int32, _L)
    zero = jnp.zeros((_L,), jnp.float32)

    def compute(c):
        par = lax.rem(c, _NBUF)
        po = par * _CHUNK
        off = c * _CHUNK

        def group_body(g, _):
            def row_body(i, res):
                r = g * _L + i
                accs = [zero for _ in range(1 + NEG)]
                for q in range(DIM // _L):
                    sl = pl.ds(q * _L, _L)
                    t = tgt_v[po + r, sl]
                    accs[0] = accs[0] + t * ctx_v[po + r, sl]
                    for k in range(NEG):
                        accs[1 + k] = accs[1 + k] + t * neg_v[
                            po * NEG + k * _CHUNK + r, sl]
                m = lanes == i
                return tuple(
                    jnp.where(m, jnp.sum(a), res[d]) for d, a in enumerate(accs)
                )

            res = lax.fori_loop(0, _L, row_body, (zero,) * (1 + NEG))
            v = off + g * _L
            pos_s_v[0, pl.ds(v, _L)] = res[0]
            for k in range(NEG):
                neg_s_v[k, pl.ds(v, _L)] = res[1 + k]
            return 0

        lax.fori_loop(0, _NGROUPS, group_body, 0)

    for c0 in range(_NBUF - 1):
        fire(c0)

    def chunk_body(c, _):
        @pl.when(c + _NBUF - 1 < _NCHUNKS)
        def _():
            fire(c + _NBUF - 1)

        wait(c)
        compute(c)
        return 0

    lax.fori_loop(0, _NCHUNKS, chunk_body, 0)

    pltpu.sync_copy(pos_s_v, pos_out_hbm.at[:, pl.ds(base, _BPW)])
    pltpu.sync_copy(neg_s_v, neg_out_hbm.at[:, pl.ds(base, _BPW)])


def _loss_body(pos_ref, neg_ref, pos_loss_ref, neg_loss_ref):
    p = pos_ref[...]
    lsp = jnp.minimum(p, 0.0) - jnp.log1p(jnp.exp(-jnp.abs(p)))
    pos_loss_ref[0, 0] = -jnp.sum(lsp) / float(BATCH)
    x = -neg_ref[...]
    lsn = jnp.minimum(x, 0.0) - jnp.log1p(jnp.exp(-jnp.abs(x)))
    neg_loss_ref[0, 0] = -jnp.sum(lsn) / float(BATCH * NEG)


_loss_call = pl.pallas_call(
    _loss_body,
    out_shape=(
        jax.ShapeDtypeStruct((1, 1), jnp.float32),
        jax.ShapeDtypeStruct((1, 1), jnp.float32),
    ),
    out_specs=(
        pl.BlockSpec(memory_space=pltpu.SMEM),
        pl.BlockSpec(memory_space=pltpu.SMEM),
    ),
)


def kernel(target_words, context_words, negative_words, input_emb, output_emb):
    tw = target_words.astype(jnp.int32)
    cw = context_words.astype(jnp.int32)
    nw = negative_words.astype(jnp.int32).T
    pos_scores, neg_scores = _sc_scores(tw, cw, nw, input_emb, output_emb)
    pos_loss, neg_loss = _loss_call(pos_scores, neg_scores)
    return (pos_loss[0, 0], neg_loss[0, 0])


# speedup vs baseline: 1.0075x; 1.0075x over previous
import functools

import jax
import jax.numpy as jnp
from jax import lax
from jax.experimental import pallas as pl
from jax.experimental.pallas import tpu as pltpu
from jax.experimental.pallas import tpu_sc as plsc

VOCAB = 100000
DIM = 128
BATCH = 16384
NEG = 5

_info = plsc.get_sparse_core_info()
_NC, _NS, _L = _info.num_cores, _info.num_subcores, _info.num_lanes
_NW = _NC * _NS
_BPW = BATCH // _NW
_CHUNK = 32
_NBUF = 3
_NCHUNKS = _BPW // _CHUNK
_NGROUPS = _CHUNK // _L

_mesh = plsc.VectorSubcoreMesh(core_axis_name="c", subcore_axis_name="s")


@functools.partial(
    pl.kernel,
    mesh=_mesh,
    compiler_params=pltpu.CompilerParams(needs_layout_passes=False),
    out_type=(
        jax.ShapeDtypeStruct((1, BATCH), jnp.float32),
        jax.ShapeDtypeStruct((NEG, BATCH), jnp.float32),
    ),
    scratch_types=[
        pltpu.VMEM((_BPW,), jnp.int32),
        pltpu.VMEM((_BPW,), jnp.int32),
        pltpu.VMEM((NEG, _BPW), jnp.int32),
        pltpu.VMEM((_NBUF * _CHUNK, DIM), jnp.float32),
        pltpu.VMEM((_NBUF * _CHUNK, DIM), jnp.float32),
        pltpu.VMEM((_NBUF * _CHUNK * NEG, DIM), jnp.float32),
        pltpu.VMEM((1, _BPW), jnp.float32),
        pltpu.VMEM((NEG, _BPW), jnp.float32),
        pltpu.SemaphoreType.DMA((_NBUF,)),
    ],
)
def _sc_scores(tgt_idx_hbm, ctx_idx_hbm, neg_idx_hbm, in_emb_hbm, out_emb_hbm,
               pos_out_hbm, neg_out_hbm,
               idx_t_v, idx_c_v, idx_n_v,
               tgt_v, ctx_v, neg_v,
               pos_s_v, neg_s_v, sems):
    wid = lax.axis_index("s") * _NC + lax.axis_index("c")
    base = wid * _BPW

    pltpu.sync_copy(tgt_idx_hbm.at[pl.ds(base, _BPW)], idx_t_v)
    pltpu.sync_copy(ctx_idx_hbm.at[pl.ds(base, _BPW)], idx_c_v)
    pltpu.sync_copy(neg_idx_hbm.at[:, pl.ds(base, _BPW)], idx_n_v)

    def fire(c):
        par = lax.rem(c, _NBUF)
        po = par * _CHUNK
        o = c * _CHUNK
        s_b = sems.at[par]
        pltpu.async_copy(
            in_emb_hbm.at[idx_t_v.at[pl.ds(o, _CHUNK)]],
            tgt_v.at[pl.ds(po, _CHUNK), :], s_b)
        pltpu.async_copy(
            out_emb_hbm.at[idx_c_v.at[pl.ds(o, _CHUNK)]],
            ctx_v.at[pl.ds(po, _CHUNK), :], s_b)
        for k in range(NEG):
            pltpu.async_copy(
                out_emb_hbm.at[idx_n_v.at[k, pl.ds(o, _CHUNK)]],
                neg_v.at[pl.ds(po * NEG + k * _CHUNK, _CHUNK), :], s_b)

    def wait(c):
        par = lax.rem(c, _NBUF)
        po = par * _CHUNK
        s_b = sems.at[par]
        pltpu.make_async_copy(
            in_emb_hbm.at[idx_t_v.at[pl.ds(0, _CHUNK)]],
            tgt_v.at[pl.ds(po, _CHUNK), :], s_b).wait()
        pltpu.make_async_copy(
            out_emb_hbm.at[idx_c_v.at[pl.ds(0, _CHUNK)]],
            ctx_v.at[pl.ds(po, _CHUNK), :], s_b).wait()
        for k in range(NEG):
            pltpu.make_async_copy(
                out_emb_hbm.at[idx_n_v.at[k, pl.ds(0, _CHUNK)]],
                neg_v.at[pl.ds(po * NEG + k * _CHUNK, _CHUNK), :], s_b).wait()

    lanes = lax.iota(jnp.int32, _L)
    zero = jnp.zeros((_L,), jnp.float32)

    def compute(c):
        par = lax.rem(c, _NBUF)
        po = par * _CHUNK
        off = c * _CHUNK

        def group_body(g, _):
            def row_body(i, res):
                r = g * _L + i
                accs = [zero for _ in range(1 + NEG)]
                for q in range(DIM // _L):
                    sl = pl.ds(q * _L, _L)
                    t = tgt_v[po + r, sl]
                    accs[0] = accs[0] + t * ctx_v[po + r, sl]
                    for k in range(NEG):
                        accs[1 + k] = accs[1 + k] + t * neg_v[
                            po * NEG + k * _CHUNK + r, sl]
                m = lanes == i
                return tuple(
                    jnp.where(m, jnp.sum(a), res[d]) for d, a in enumerate(accs)
                )

            res = lax.fori_loop(0, _L, row_body, (zero,) * (1 + NEG))
            v = off + g * _L
            pos_s_v[0, pl.ds(v, _L)] = res[0]
            for k in range(NEG):
                neg_s_v[k, pl.ds(v, _L)] = res[1 + k]
            return 0

        lax.fori_loop(0, _NGROUPS, group_body, 0)

    for c0 in range(_NBUF - 1):
        fire(c0)

    def chunk_body(c, _):
        @pl.when(c + _NBUF - 1 < _NCHUNKS)
        def _():
            fire(c + _NBUF - 1)

        wait(c)
        compute(c)
        return 0

    lax.fori_loop(0, _NCHUNKS, chunk_body, 0)

    pltpu.sync_copy(pos_s_v, pos_out_hbm.at[:, pl.ds(base, _BPW)])
    pltpu.sync_copy(neg_s_v, neg_out_hbm.at[:, pl.ds(base, _BPW)])


def _loss_body(pos_ref, neg_ref, pos_loss_ref, neg_loss_ref):
    p = pos_ref[...]
    lsp = jnp.minimum(p, 0.0) - jnp.log1p(jnp.exp(-jnp.abs(p)))
    pos_loss_ref[0, 0] = -jnp.sum(lsp) / float(BATCH)
    x = -neg_ref[...]
    lsn = jnp.minimum(x, 0.0) - jnp.log1p(jnp.exp(-jnp.abs(x)))
    neg_loss_ref[0, 0] = -jnp.sum(lsn) / float(BATCH * NEG)


_loss_call = pl.pallas_call(
    _loss_body,
    out_shape=(
        jax.ShapeDtypeStruct((1, 1), jnp.float32),
        jax.ShapeDtypeStruct((1, 1), jnp.float32),
    ),
    out_specs=(
        pl.BlockSpec(memory_space=pltpu.SMEM),
        pl.BlockSpec(memory_space=pltpu.SMEM),
    ),
)


def kernel(target_words, context_words, negative_words, input_emb, output_emb):
    tw = target_words.astype(jnp.int32)
    cw = context_words.astype(jnp.int32)
    nw = negative_words.astype(jnp.int32).T
    pos_scores, neg_scores = _sc_scores(tw, cw, nw, input_emb, output_emb)
    pos_loss, neg_loss = _loss_call(pos_scores, neg_scores)
    return (pos_loss[0, 0], neg_loss[0, 0])


# speedup vs baseline: 1.0083x; 1.0008x over previous
import functools

import jax
import jax.numpy as jnp
from jax import lax
from jax.experimental import pallas as pl
from jax.experimental.pallas import tpu as pltpu
from jax.experimental.pallas import tpu_sc as plsc

VOCAB = 100000
DIM = 128
BATCH = 16384
NEG = 5

_info = plsc.get_sparse_core_info()
_NC, _NS, _L = _info.num_cores, _info.num_subcores, _info.num_lanes
_NW = _NC * _NS
_BPW = BATCH // _NW
_CHUNK = 32
_NBUF = 3
_NCHUNKS = _BPW // _CHUNK
_NGROUPS = _CHUNK // _L

_mesh = plsc.VectorSubcoreMesh(core_axis_name="c", subcore_axis_name="s")


@functools.partial(
    pl.kernel,
    mesh=_mesh,
    compiler_params=pltpu.CompilerParams(needs_layout_passes=False),
    out_type=jax.ShapeDtypeStruct((NEG + 1, BATCH), jnp.float32),
    scratch_types=[
        pltpu.VMEM((_BPW,), jnp.int32),
        pltpu.VMEM((_BPW,), jnp.int32),
        pltpu.VMEM((NEG, _BPW), jnp.int32),
        pltpu.VMEM((_NBUF * _CHUNK, DIM), jnp.float32),
        pltpu.VMEM((_NBUF * _CHUNK, DIM), jnp.float32),
        pltpu.VMEM((_NBUF * _CHUNK * NEG, DIM), jnp.float32),
        pltpu.VMEM((NEG + 1, _BPW), jnp.float32),
        pltpu.SemaphoreType.DMA((_NBUF,)),
    ],
)
def _sc_scores(tgt_idx_hbm, ctx_idx_hbm, neg_idx_hbm, in_emb_hbm, out_emb_hbm,
               scores_out_hbm,
               idx_t_v, idx_c_v, idx_n_v,
               tgt_v, ctx_v, neg_v,
               scores_v, sems):
    wid = lax.axis_index("s") * _NC + lax.axis_index("c")
    base = wid * _BPW

    pltpu.sync_copy(tgt_idx_hbm.at[pl.ds(base, _BPW)], idx_t_v)
    pltpu.sync_copy(ctx_idx_hbm.at[pl.ds(base, _BPW)], idx_c_v)
    pltpu.sync_copy(neg_idx_hbm.at[:, pl.ds(base, _BPW)], idx_n_v)

    def fire(c):
        par = lax.rem(c, _NBUF)
        po = par * _CHUNK
        o = c * _CHUNK
        s_b = sems.at[par]
        pltpu.async_copy(
            in_emb_hbm.at[idx_t_v.at[pl.ds(o, _CHUNK)]],
            tgt_v.at[pl.ds(po, _CHUNK), :], s_b)
        pltpu.async_copy(
            out_emb_hbm.at[idx_c_v.at[pl.ds(o, _CHUNK)]],
            ctx_v.at[pl.ds(po, _CHUNK), :], s_b)
        for k in range(NEG):
            pltpu.async_copy(
                out_emb_hbm.at[idx_n_v.at[k, pl.ds(o, _CHUNK)]],
                neg_v.at[pl.ds(po * NEG + k * _CHUNK, _CHUNK), :], s_b)

    def wait(c):
        par = lax.rem(c, _NBUF)
        po = par * _CHUNK
        s_b = sems.at[par]
        pltpu.make_async_copy(
            in_emb_hbm.at[idx_t_v.at[pl.ds(0, _CHUNK)]],
            tgt_v.at[pl.ds(po, _CHUNK), :], s_b).wait()
        pltpu.make_async_copy(
            out_emb_hbm.at[idx_c_v.at[pl.ds(0, _CHUNK)]],
            ctx_v.at[pl.ds(po, _CHUNK), :], s_b).wait()
        for k in range(NEG):
            pltpu.make_async_copy(
                out_emb_hbm.at[idx_n_v.at[k, pl.ds(0, _CHUNK)]],
                neg_v.at[pl.ds(po * NEG + k * _CHUNK, _CHUNK), :], s_b).wait()

    lanes = lax.iota(jnp.int32, _L)
    zero = jnp.zeros((_L,), jnp.float32)

    def compute(c):
        par = lax.rem(c, _NBUF)
        po = par * _CHUNK
        off = c * _CHUNK

        def group_body(g, _):
            def row_body(i, res):
                r = g * _L + i
                accs = [zero for _ in range(1 + NEG)]
                for q in range(DIM // _L):
                    sl = pl.ds(q * _L, _L)
                    t = tgt_v[po + r, sl]
                    accs[0] = accs[0] + t * ctx_v[po + r, sl]
                    for k in range(NEG):
                        accs[1 + k] = accs[1 + k] + t * neg_v[
                            po * NEG + k * _CHUNK + r, sl]
                m = lanes == i
                return tuple(
                    jnp.where(m, jnp.sum(a), res[d]) for d, a in enumerate(accs)
                )

            res = lax.fori_loop(0, _L, row_body, (zero,) * (1 + NEG))
            v = off + g * _L
            scores_v[NEG, pl.ds(v, _L)] = res[0]
            for k in range(NEG):
                scores_v[k, pl.ds(v, _L)] = res[1 + k]
            return 0

        lax.fori_loop(0, _NGROUPS, group_body, 0)

    for c0 in range(_NBUF - 1):
        fire(c0)

    def chunk_body(c, _):
        @pl.when(c + _NBUF - 1 < _NCHUNKS)
        def _():
            fire(c + _NBUF - 1)

        wait(c)
        compute(c)
        return 0

    lax.fori_loop(0, _NCHUNKS, chunk_body, 0)

    pltpu.sync_copy(scores_v, scores_out_hbm.at[:, pl.ds(base, _BPW)])


def _loss_body(scores_ref, pos_loss_ref, neg_loss_ref):
    sc = scores_ref[...]
    p = sc[NEG:, :]
    lsp = jnp.minimum(p, 0.0) - jnp.log1p(jnp.exp(-jnp.abs(p)))
    pos_loss_ref[0, 0] = -jnp.sum(lsp) / float(BATCH)
    x = -sc[:NEG, :]
    lsn = jnp.minimum(x, 0.0) - jnp.log1p(jnp.exp(-jnp.abs(x)))
    neg_loss_ref[0, 0] = -jnp.sum(lsn) / float(BATCH * NEG)


_loss_call = pl.pallas_call(
    _loss_body,
    out_shape=(
        jax.ShapeDtypeStruct((1, 1), jnp.float32),
        jax.ShapeDtypeStruct((1, 1), jnp.float32),
    ),
    out_specs=(
        pl.BlockSpec(memory_space=pltpu.SMEM),
        pl.BlockSpec(memory_space=pltpu.SMEM),
    ),
)


def kernel(target_words, context_words, negative_words, input_emb, output_emb):
    tw = target_words.astype(jnp.int32)
    cw = context_words.astype(jnp.int32)
    nw = negative_words.astype(jnp.int32).T
    scores = _sc_scores(tw, cw, nw, input_emb, output_emb)
    pos_loss, neg_loss = _loss_call(scores)
    return (pos_loss[0, 0], neg_loss[0, 0])


# speedup vs baseline: 1.0313x; 1.0228x over previous
import functools

import jax
import jax.numpy as jnp
from jax import lax
from jax.experimental import pallas as pl
from jax.experimental.pallas import tpu as pltpu
from jax.experimental.pallas import tpu_sc as plsc

VOCAB = 100000
DIM = 128
BATCH = 16384
NEG = 5

_info = plsc.get_sparse_core_info()
_NC, _NS, _L = _info.num_cores, _info.num_subcores, _info.num_lanes
_NW = _NC * _NS
_BPW = BATCH // _NW
_CHUNK = 16
_NBUF = 6
_NCHUNKS = _BPW // _CHUNK
_NGROUPS = _CHUNK // _L

_mesh = plsc.VectorSubcoreMesh(core_axis_name="c", subcore_axis_name="s")


@functools.partial(
    pl.kernel,
    mesh=_mesh,
    compiler_params=pltpu.CompilerParams(needs_layout_passes=False),
    out_type=jax.ShapeDtypeStruct((NEG + 1, BATCH), jnp.float32),
    scratch_types=[
        pltpu.VMEM((_BPW,), jnp.int32),
        pltpu.VMEM((_BPW,), jnp.int32),
        pltpu.VMEM((NEG, _BPW), jnp.int32),
        pltpu.VMEM((_NBUF * _CHUNK, DIM), jnp.float32),
        pltpu.VMEM((_NBUF * _CHUNK, DIM), jnp.float32),
        pltpu.VMEM((_NBUF * _CHUNK * NEG, DIM), jnp.float32),
        pltpu.VMEM((NEG + 1, _BPW), jnp.float32),
        pltpu.SemaphoreType.DMA((_NBUF,)),
    ],
)
def _sc_scores(tgt_idx_hbm, ctx_idx_hbm, neg_idx_hbm, in_emb_hbm, out_emb_hbm,
               scores_out_hbm,
               idx_t_v, idx_c_v, idx_n_v,
               tgt_v, ctx_v, neg_v,
               scores_v, sems):
    wid = lax.axis_index("s") * _NC + lax.axis_index("c")
    base = wid * _BPW

    pltpu.sync_copy(tgt_idx_hbm.at[pl.ds(base, _BPW)], idx_t_v)
    pltpu.sync_copy(ctx_idx_hbm.at[pl.ds(base, _BPW)], idx_c_v)
    pltpu.sync_copy(neg_idx_hbm.at[:, pl.ds(base, _BPW)], idx_n_v)

    def fire(c):
        par = lax.rem(c, _NBUF)
        po = par * _CHUNK
        o = c * _CHUNK
        s_b = sems.at[par]
        pltpu.async_copy(
            in_emb_hbm.at[idx_t_v.at[pl.ds(o, _CHUNK)]],
            tgt_v.at[pl.ds(po, _CHUNK), :], s_b)
        pltpu.async_copy(
            out_emb_hbm.at[idx_c_v.at[pl.ds(o, _CHUNK)]],
            ctx_v.at[pl.ds(po, _CHUNK), :], s_b)
        for k in range(NEG):
            pltpu.async_copy(
                out_emb_hbm.at[idx_n_v.at[k, pl.ds(o, _CHUNK)]],
                neg_v.at[pl.ds(po * NEG + k * _CHUNK, _CHUNK), :], s_b)

    def wait(c):
        par = lax.rem(c, _NBUF)
        po = par * _CHUNK
        s_b = sems.at[par]
        pltpu.make_async_copy(
            in_emb_hbm.at[idx_t_v.at[pl.ds(0, _CHUNK)]],
            tgt_v.at[pl.ds(po, _CHUNK), :], s_b).wait()
        pltpu.make_async_copy(
            out_emb_hbm.at[idx_c_v.at[pl.ds(0, _CHUNK)]],
            ctx_v.at[pl.ds(po, _CHUNK), :], s_b).wait()
        for k in range(NEG):
            pltpu.make_async_copy(
                out_emb_hbm.at[idx_n_v.at[k, pl.ds(0, _CHUNK)]],
                neg_v.at[pl.ds(po * NEG + k * _CHUNK, _CHUNK), :], s_b).wait()

    lanes = lax.iota(jnp.int32, _L)
    zero = jnp.zeros((_L,), jnp.float32)

    def compute(c):
        par = lax.rem(c, _NBUF)
        po = par * _CHUNK
        off = c * _CHUNK

        def group_body(g, _):
            def row_body(i, res):
                r = g * _L + i
                accs = [zero for _ in range(1 + NEG)]
                for q in range(DIM // _L):
                    sl = pl.ds(q * _L, _L)
                    t = tgt_v[po + r, sl]
                    accs[0] = accs[0] + t * ctx_v[po + r, sl]
                    for k in range(NEG):
                        accs[1 + k] = accs[1 + k] + t * neg_v[
                            po * NEG + k * _CHUNK + r, sl]
                m = lanes == i
                return tuple(
                    jnp.where(m, jnp.sum(a), res[d]) for d, a in enumerate(accs)
                )

            res = lax.fori_loop(0, _L, row_body, (zero,) * (1 + NEG))
            v = off + g * _L
            scores_v[NEG, pl.ds(v, _L)] = res[0]
            for k in range(NEG):
                scores_v[k, pl.ds(v, _L)] = res[1 + k]
            return 0

        lax.fori_loop(0, _NGROUPS, group_body, 0)

    for c0 in range(_NBUF - 1):
        fire(c0)

    def chunk_body(c, _):
        @pl.when(c + _NBUF - 1 < _NCHUNKS)
        def _():
            fire(c + _NBUF - 1)

        wait(c)
        compute(c)
        return 0

    lax.fori_loop(0, _NCHUNKS, chunk_body, 0)

    pltpu.sync_copy(scores_v, scores_out_hbm.at[:, pl.ds(base, _BPW)])


def _loss_body(scores_ref, pos_loss_ref, neg_loss_ref):
    sc = scores_ref[...]
    p = sc[NEG:, :]
    lsp = jnp.minimum(p, 0.0) - jnp.log1p(jnp.exp(-jnp.abs(p)))
    pos_loss_ref[0, 0] = -jnp.sum(lsp) / float(BATCH)
    x = -sc[:NEG, :]
    lsn = jnp.minimum(x, 0.0) - jnp.log1p(jnp.exp(-jnp.abs(x)))
    neg_loss_ref[0, 0] = -jnp.sum(lsn) / float(BATCH * NEG)


_loss_call = pl.pallas_call(
    _loss_body,
    out_shape=(
        jax.ShapeDtypeStruct((1, 1), jnp.float32),
        jax.ShapeDtypeStruct((1, 1), jnp.float32),
    ),
    out_specs=(
        pl.BlockSpec(memory_space=pltpu.SMEM),
        pl.BlockSpec(memory_space=pltpu.SMEM),
    ),
)


def kernel(target_words, context_words, negative_words, input_emb, output_emb):
    tw = target_words.astype(jnp.int32)
    cw = context_words.astype(jnp.int32)
    nw = negative_words.astype(jnp.int32).T
    scores = _sc_scores(tw, cw, nw, input_emb, output_emb)
    pos_loss, neg_loss = _loss_call(scores)
    return (pos_loss[0, 0], neg_loss[0, 0])
